# baseline (speedup 1.0000x reference)
import functools

import jax
import jax.numpy as jnp
from jax import lax
from jax.experimental import pallas as pl
from jax.experimental.pallas import tpu as pltpu
from jax.experimental.pallas import tpu_sc as plsc

BATCH = 16384
FIELDS = 100
NC = 2
NS = 16
NW = NC * NS
ROWS_PER_W = BATCH // NW
GROUPS = ROWS_PER_W // 16
TBL = 1000448
NCHUNK = 2
FPC = FIELDS // NCHUNK


@functools.partial(
    pl.kernel,
    out_type=jax.ShapeDtypeStruct((BATCH,), jnp.float32),
    mesh=plsc.VectorSubcoreMesh(core_axis_name="c", subcore_axis_name="s"),
    compiler_params=pltpu.CompilerParams(needs_layout_passes=False),
    scratch_types=[
        pltpu.VMEM((FIELDS * ROWS_PER_W,), jnp.int32),
        pltpu.VMEM((FIELDS * ROWS_PER_W,), jnp.float32),
        pltpu.VMEM((ROWS_PER_W,), jnp.float32),
        pltpu.SemaphoreType.DMA,
        pltpu.SemaphoreType.DMA,
    ],
)
def _wide_sum(xt_hbm, table_hbm, out_hbm, xv, vv, ov, semi, semg):
    cid = lax.axis_index("c")
    sid = lax.axis_index("s")
    wid = sid * NC + cid
    base = wid * ROWS_PER_W

    stage = [
        pltpu.async_copy(
            xt_hbm.at[f, pl.ds(base, ROWS_PER_W)],
            xv.at[pl.ds(f * ROWS_PER_W, ROWS_PER_W)],
            semi,
        )
        for f in range(FIELDS)
    ]
    for c in stage:
        c.wait()

    nc = FPC * ROWS_PER_W
    gathers = [
        pltpu.async_copy(
            table_hbm.at[xv.at[pl.ds(k * nc, nc)]],
            vv.at[pl.ds(k * nc, nc)],
            semg,
        )
        for k in range(NCHUNK)
    ]

    for k in range(NCHUNK):
        gathers[k].wait()

        def chunk_body(g, _, k=k):
            r0 = g * 16
            accs = [jnp.zeros((16,), jnp.float32) for _ in range(4)]
            for f in range(k * FPC, (k + 1) * FPC):
                accs[f % 4] = accs[f % 4] + vv[pl.ds(f * ROWS_PER_W + r0, 16)]
            tot = (accs[0] + accs[1]) + (accs[2] + accs[3])
            if k == 0:
                ov[pl.ds(r0, 16)] = tot
            else:
                ov[pl.ds(r0, 16)] = ov[pl.ds(r0, 16)] + tot
            return _

        lax.fori_loop(0, GROUPS, chunk_body, None)

    pltpu.sync_copy(ov, out_hbm.at[pl.ds(base, ROWS_PER_W)])


def kernel(X, table):
    xt = X.T
    t_flat = jnp.pad(table, ((0, TBL - 1000001), (0, 0))).reshape(-1)
    out = _wide_sum(xt, t_flat)
    return out.reshape(BATCH, 1)


# speedup vs baseline: 1.0026x; 1.0026x over previous
import functools

import jax
import jax.numpy as jnp
from jax import lax
from jax.experimental import pallas as pl
from jax.experimental.pallas import tpu as pltpu
from jax.experimental.pallas import tpu_sc as plsc

BATCH = 16384
FIELDS = 100
NC = 2
NS = 16
NW = NC * NS
ROWS_PER_W = BATCH // NW
GROUPS = ROWS_PER_W // 16
TBL = 1000448
NCHUNK = 4
FPC = FIELDS // NCHUNK


@functools.partial(
    pl.kernel,
    out_type=jax.ShapeDtypeStruct((BATCH,), jnp.float32),
    mesh=plsc.VectorSubcoreMesh(core_axis_name="c", subcore_axis_name="s"),
    compiler_params=pltpu.CompilerParams(needs_layout_passes=False),
    scratch_types=[
        pltpu.VMEM((FIELDS * ROWS_PER_W,), jnp.int32),
        pltpu.VMEM((FIELDS * ROWS_PER_W,), jnp.float32),
        pltpu.VMEM((ROWS_PER_W,), jnp.float32),
        pltpu.SemaphoreType.DMA,
        pltpu.SemaphoreType.DMA,
    ],
)
def _wide_sum(xt_hbm, table_hbm, out_hbm, xv, vv, ov, semi, semg):
    cid = lax.axis_index("c")
    sid = lax.axis_index("s")
    wid = sid * NC + cid
    base = wid * ROWS_PER_W

    stage = [
        pltpu.async_copy(
            xt_hbm.at[f, pl.ds(base, ROWS_PER_W)],
            xv.at[pl.ds(f * ROWS_PER_W, ROWS_PER_W)],
            semi,
        )
        for f in range(FIELDS)
    ]
    for c in stage:
        c.wait()

    nc = FPC * ROWS_PER_W
    gathers = [
        pltpu.async_copy(
            table_hbm.at[xv.at[pl.ds(k * nc, nc)]],
            vv.at[pl.ds(k * nc, nc)],
            semg,
        )
        for k in range(NCHUNK)
    ]

    for k in range(NCHUNK):
        gathers[k].wait()

        def chunk_body(g, _, k=k):
            r0 = g * 16
            accs = [jnp.zeros((16,), jnp.float32) for _ in range(4)]
            for f in range(k * FPC, (k + 1) * FPC):
                accs[f % 4] = accs[f % 4] + vv[pl.ds(f * ROWS_PER_W + r0, 16)]
            tot = (accs[0] + accs[1]) + (accs[2] + accs[3])
            if k == 0:
                ov[pl.ds(r0, 16)] = tot
            else:
                ov[pl.ds(r0, 16)] = ov[pl.ds(r0, 16)] + tot
            return _

        lax.fori_loop(0, GROUPS, chunk_body, None)

    pltpu.sync_copy(ov, out_hbm.at[pl.ds(base, ROWS_PER_W)])


def kernel(X, table):
    xt = X.T
    t_flat = jnp.pad(table, ((0, TBL - 1000001), (0, 0))).reshape(-1)
    out = _wide_sum(xt, t_flat)
    return out.reshape(BATCH, 1)
